# baseline (speedup 1.0000x reference)
import functools

import jax
import jax.numpy as jnp
from jax import lax
from jax.experimental import pallas as pl
from jax.experimental.pallas import tpu as pltpu
from jax.experimental.pallas import tpu_sc as plsc
from jax.experimental.layout import Layout, with_layout_constraint

BATCH = 16384
FIELDS = 26
EMBED_DIM = 32
N = BATCH * FIELDS
NUM_WORKERS = 32
PER_WORKER = N // NUM_WORKERS
CHUNK = 1664
NUM_CHUNKS = PER_WORKER // CHUNK
NBUF = 2

VOCAB1 = 1000001
TBLK = 16384
NBLK = (VOCAB1 + TBLK - 1) // TBLK
VPAD = NBLK * TBLK


def _repack_block(src_ref, dst_ref):
    x = src_ref[...]
    for m in range(TBLK // 512):
        xs = jnp.concatenate(
            [x[:, 512 * m + 128 * a:512 * m + 128 * a + 128] for a in range(4)],
            axis=0)
        dst_ref[128 * m:128 * (m + 1), :] = xs.T


def _repack_table(table):
    tt = table.T
    return pl.pallas_call(
        _repack_block,
        grid=(NBLK,),
        in_specs=[pl.BlockSpec((EMBED_DIM, TBLK), lambda k: (0, k))],
        out_specs=pl.BlockSpec((TBLK // 4, 128), lambda k: (k, 0)),
        out_shape=jax.ShapeDtypeStruct((VPAD // 4, 128), jnp.float32),
    )(tt)


def _pair_xpose_block(src_ref, dst_ref):
    for h in range(2):
        for m in range(BATCH // 256):
            dst_ref[:, (BATCH // 2) * h + 128 * m:
                    (BATCH // 2) * h + 128 * (m + 1)] = (
                src_ref[128 * m:128 * (m + 1), 64 * h:64 * (h + 1)].T)


def _pair_xpose(sc_mid):
    return pl.pallas_call(
        _pair_xpose_block,
        grid=(FIELDS // 2,),
        in_specs=[pl.BlockSpec((BATCH // 2, 128), lambda p: (p, 0))],
        out_specs=pl.BlockSpec((64, BATCH), lambda p: (p, 0)),
        out_shape=jax.ShapeDtypeStruct(
            (FIELDS * EMBED_DIM, BATCH), jnp.float32),
    )(sc_mid)


def kernel(indices, table):
    num_item = table.shape[0] - 1
    it = indices.T
    idx_perm = jnp.transpose(
        it.reshape(FIELDS // 2, 2, 2, BATCH // 2), (0, 3, 2, 1)).reshape(N)
    idx_perm = idx_perm.astype(jnp.int32)

    t128 = _repack_table(table)
    table_lin = with_layout_constraint(
        t128.reshape(VPAD, EMBED_DIM),
        Layout(major_to_minor=(0, 1), tiling=((8,), (1024,))))

    mesh = plsc.VectorSubcoreMesh(core_axis_name="c", subcore_axis_name="s")

    @functools.partial(
        pl.kernel,
        mesh=mesh,
        out_type=jax.ShapeDtypeStruct((N, EMBED_DIM), jnp.float32),
        scratch_types=[
            pltpu.VMEM((NBUF, CHUNK), jnp.int32),
            pltpu.VMEM((NBUF, CHUNK, EMBED_DIM), jnp.float32),
            pltpu.SemaphoreType.DMA((NBUF,)),
            pltpu.SemaphoreType.DMA((NBUF,)),
        ],
        compiler_params=pltpu.CompilerParams(use_tc_tiling_on_sc=False),
    )
    def gather_kernel(idx_hbm, table_hbm, out_hbm, idx_v, rows_v, gsem, ssem):
        wid = lax.axis_index("s") * 2 + lax.axis_index("c")
        base = wid * PER_WORKER

        def start_gather(c):
            b = c % NBUF
            off = base + c * CHUNK
            pltpu.sync_copy(idx_hbm.at[pl.ds(off, CHUNK)], idx_v.at[b])

            def remap_body(i, carry):
                v = idx_v[b, pl.ds(i * 16, 16)]
                v = jnp.where(v == -1, num_item, v)
                p = ((v & ~511) | ((v & 127) << 2)
                     | ((v >> 7) & 3))
                idx_v[b, pl.ds(i * 16, 16)] = p
                return carry

            lax.fori_loop(0, CHUNK // 16, remap_body, 0, unroll=8)
            return pltpu.async_copy(
                table_hbm.at[idx_v.at[b]], rows_v.at[b], gsem.at[b])

        def start_store(c):
            b = c % NBUF
            off = base + c * CHUNK
            return pltpu.async_copy(
                rows_v.at[b], out_hbm.at[pl.ds(off, CHUNK)], ssem.at[b])

        gh = [None] * NUM_CHUNKS
        sh = [None] * NUM_CHUNKS
        for c in range(NUM_CHUNKS):
            if c >= NBUF:
                sh[c - NBUF].wait()
            gh[c] = start_gather(c)
            if c >= 1:
                gh[c - 1].wait()
                sh[c - 1] = start_store(c - 1)
        gh[NUM_CHUNKS - 1].wait()
        sh[NUM_CHUNKS - 1] = start_store(NUM_CHUNKS - 1)
        sh[NUM_CHUNKS - 2].wait()
        sh[NUM_CHUNKS - 1].wait()

    out = gather_kernel(idx_perm, table_lin)
    sc_mid = out.reshape(N // 4, 4 * EMBED_DIM)
    out_t = _pair_xpose(sc_mid)
    return out_t.T


# speedup vs baseline: 2.2960x; 2.2960x over previous
import functools

import jax
import jax.numpy as jnp
from jax import lax
from jax.experimental import pallas as pl
from jax.experimental.pallas import tpu as pltpu
from jax.experimental.pallas import tpu_sc as plsc
from jax.experimental.layout import Layout, with_layout_constraint

BATCH = 16384
FIELDS = 26
EMBED_DIM = 32
N = BATCH * FIELDS
NUM_WORKERS = 32
PER_WORKER = N // NUM_WORKERS
CHUNK = 1664
NUM_CHUNKS = PER_WORKER // CHUNK
NBUF = 2

VOCAB1 = 1000001
TBLK = 16384
NBLK = (VOCAB1 + TBLK - 1) // TBLK
VPAD = NBLK * TBLK


def _repack_block(src_ref, dst_ref):
    x = src_ref[...]
    for m in range(TBLK // 512):
        xs = jnp.concatenate(
            [x[:, 512 * m + 128 * a:512 * m + 128 * a + 128] for a in range(4)],
            axis=0)
        dst_ref[128 * m:128 * (m + 1), :] = xs.T


def _repack_table(table):
    tt = table.T
    return pl.pallas_call(
        _repack_block,
        grid=(NBLK,),
        in_specs=[pl.BlockSpec((EMBED_DIM, TBLK), lambda k: (0, k))],
        out_specs=pl.BlockSpec((TBLK // 4, 128), lambda k: (k, 0)),
        out_shape=jax.ShapeDtypeStruct((VPAD // 4, 128), jnp.float32),
    )(tt)


def _pair_xpose_block(src_ref, dst_ref):
    for h in range(2):
        for m in range(BATCH // 256):
            dst_ref[:, (BATCH // 2) * h + 128 * m:
                    (BATCH // 2) * h + 128 * (m + 1)] = (
                src_ref[128 * m:128 * (m + 1), 64 * h:64 * (h + 1)].T)


def _pair_xpose(sc_mid):
    return pl.pallas_call(
        _pair_xpose_block,
        grid=(FIELDS // 2,),
        in_specs=[pl.BlockSpec((BATCH // 2, 128), lambda p: (p, 0))],
        out_specs=pl.BlockSpec((64, BATCH), lambda p: (p, 0)),
        out_shape=jax.ShapeDtypeStruct(
            (FIELDS * EMBED_DIM, BATCH), jnp.float32),
    )(sc_mid)


def kernel(indices, table):
    num_item = table.shape[0] - 1
    idx_flat = indices.reshape(-1).astype(jnp.int32)

    t128 = _repack_table(table)
    table_lin = with_layout_constraint(
        t128.reshape(VPAD, EMBED_DIM),
        Layout(major_to_minor=(0, 1), tiling=((8,), (1024,))))

    mesh = plsc.VectorSubcoreMesh(core_axis_name="c", subcore_axis_name="s")

    @functools.partial(
        pl.kernel,
        mesh=mesh,
        out_type=jax.ShapeDtypeStruct((N, EMBED_DIM), jnp.float32),
        scratch_types=[
            pltpu.VMEM((NBUF, CHUNK), jnp.int32),
            pltpu.VMEM((NBUF, CHUNK), jnp.int32),
            pltpu.VMEM((NBUF, CHUNK, EMBED_DIM), jnp.float32),
            pltpu.SemaphoreType.DMA((NBUF,)),
            pltpu.SemaphoreType.DMA((NBUF,)),
            pltpu.SemaphoreType.DMA((NBUF,)),
        ],
        compiler_params=pltpu.CompilerParams(use_tc_tiling_on_sc=False),
    )
    def gather_kernel(idx_hbm, table_hbm, out_hbm,
                      pos_v, idx_v, rows_v, psem, gsem, ssem):
        wid = lax.axis_index("s") * 2 + lax.axis_index("c")
        base = wid * PER_WORKER

        def start_gather(c):
            b = c % NBUF
            off = base + c * CHUNK

            def pos_body(i, carry):
                j = off + i * 16 + lax.iota(jnp.int32, 16)
                fp = j >> 15
                beta = (j & 32767) >> 2
                g = (j >> 1) & 1
                e = j & 1
                brow = (g << 13) + beta
                pos_v[b, pl.ds(i * 16, 16)] = (
                    brow * 26 + (fp << 1) + e)
                return carry

            lax.fori_loop(0, CHUNK // 16, pos_body, 0, unroll=8)
            pltpu.async_copy(
                idx_hbm.at[pos_v.at[b]], idx_v.at[b], psem.at[b]).wait()

            def remap_body(i, carry):
                v = idx_v[b, pl.ds(i * 16, 16)]
                v = jnp.where(v == -1, num_item, v)
                p = ((v & ~511) | ((v & 127) << 2)
                     | ((v >> 7) & 3))
                idx_v[b, pl.ds(i * 16, 16)] = p
                return carry

            lax.fori_loop(0, CHUNK // 16, remap_body, 0, unroll=8)
            return pltpu.async_copy(
                table_hbm.at[idx_v.at[b]], rows_v.at[b], gsem.at[b])

        def start_store(c):
            b = c % NBUF
            off = base + c * CHUNK
            return pltpu.async_copy(
                rows_v.at[b], out_hbm.at[pl.ds(off, CHUNK)], ssem.at[b])

        gh = [None] * NUM_CHUNKS
        sh = [None] * NUM_CHUNKS
        for c in range(NUM_CHUNKS):
            if c >= NBUF:
                sh[c - NBUF].wait()
            gh[c] = start_gather(c)
            if c >= 1:
                gh[c - 1].wait()
                sh[c - 1] = start_store(c - 1)
        gh[NUM_CHUNKS - 1].wait()
        sh[NUM_CHUNKS - 1] = start_store(NUM_CHUNKS - 1)
        sh[NUM_CHUNKS - 2].wait()
        sh[NUM_CHUNKS - 1].wait()

    out = gather_kernel(idx_flat, table_lin)
    sc_mid = out.reshape(N // 4, 4 * EMBED_DIM)
    out_t = _pair_xpose(sc_mid)
    return out_t.T
